# baseline (speedup 1.0000x reference)
import functools

import jax
import jax.numpy as jnp
from jax import lax
from jax.experimental import pallas as pl
from jax.experimental.pallas import tpu as pltpu
from jax.experimental.pallas import tpu_sc as plsc

N = 10000
NPR = 640
NP16 = NPR * 16
D = 128
P = 32
E = 320000
EPT = 500000
EPT_PAD = 512000
OT_EPS = 0.1
OT_ITERS = 20
NSC = 2
NT = 16

_GDN = lax.GatherDimensionNumbers(
    offset_dims=(), collapsed_slice_dims=(0,), start_index_map=(0,))


def _lane_perm(v, idx):
    return lax.gather(v, idx[:, None], _GDN, (1,),
                      mode=lax.GatherScatterMode.PROMISE_IN_BOUNDS)


def _sc_deg_call(edge_col):
    ed = E // (NSC * NT)
    NBD = ed // 128
    TAIL = ed - NBD * 128

    mesh = plsc.VectorSubcoreMesh(core_axis_name="c", subcore_axis_name="s")

    @functools.partial(
        pl.kernel,
        out_type=jax.ShapeDtypeStruct((NSC, NP16, 16), jnp.float32),
        mesh=mesh,
        compiler_params=pltpu.CompilerParams(use_tc_tiling_on_sc=False),
        scratch_types=[
            pltpu.VMEM((ed,), jnp.int32),
            pltpu.VMEM((128, 16), jnp.float32),
            pltpu.VMEM((128,), jnp.int32),
            pltpu.VMEM((TAIL,), jnp.int32),
            pltpu.VMEM((NPR, 16), jnp.float32),
            pltpu.VMEM_SHARED((NP16, 16), jnp.float32),
        ],
    )
    def k(ec_hbm, out_hbm, cbufd, ones_rows, degidx, degidxt, bnc, deg2_sh):
        c = lax.axis_index("c")
        s = lax.axis_index("s")
        zero16 = jnp.zeros((16,), jnp.float32)
        ones16 = jnp.ones((16,), jnp.float32)

        def zrow(i, _):
            ones_rows[i, :] = ones16
            bnc[lax.rem(i, NPR), :] = zero16
            return 0
        lax.fori_loop(0, 128, zrow, 0)

        def zb(i, _):
            bnc[i, :] = zero16
            return 0
        lax.fori_loop(0, NPR, zb, 0)
        pltpu.sync_copy(bnc, deg2_sh.at[pl.ds(s * NPR, NPR)])
        plsc.subcore_barrier()

        base = c * (E // NSC) + s * ed
        pltpu.sync_copy(ec_hbm.at[pl.ds(base, ed)], cbufd)

        def degb(i, _):
            for g in range(8):
                degidx[pl.ds(g * 16, 16)] = cbufd[pl.ds(i * 128 + g * 16, 16)]
            pltpu.sync_copy(ones_rows, deg2_sh.at[degidx], add=True)
            return 0
        lax.fori_loop(0, NBD, degb, 0)
        for g in range(TAIL // 16):
            degidxt[pl.ds(g * 16, 16)] = cbufd[pl.ds(NBD * 128 + g * 16, 16)]
        pltpu.sync_copy(ones_rows.at[pl.ds(0, TAIL)], deg2_sh.at[degidxt],
                        add=True)
        plsc.subcore_barrier()

        pltpu.sync_copy(deg2_sh.at[pl.ds(s * NPR, NPR)], bnc)
        pltpu.sync_copy(bnc, out_hbm.at[c, pl.ds(s * NPR, NPR)])

    return k(edge_col)


def _dis_body(dp_ref, dis_ref):
    deg = dp_ref[0] + dp_ref[1]
    dis_ref[...] = jnp.where(deg > 0.0, jax.lax.rsqrt(jnp.maximum(deg, 1.0)), 0.0)


def _dis_call(deg_parts):
    return pl.pallas_call(
        _dis_body,
        out_shape=jax.ShapeDtypeStruct((NP16, 16), jnp.float32),
    )(deg_parts)


def _sc_aggr_call(x_half_pad, edge_row, edge_col, edge_weight, dis2):
    DH = D // 2
    ea = E // (NSC * NT)
    BA = 80

    mesh = plsc.VectorSubcoreMesh(core_axis_name="c", subcore_axis_name="s")

    @functools.partial(
        pl.kernel,
        out_type=jax.ShapeDtypeStruct((NSC, NP16, DH), jnp.float32),
        mesh=mesh,
        compiler_params=pltpu.CompilerParams(use_tc_tiling_on_sc=False),
        scratch_types=[
            pltpu.VMEM((ea,), jnp.int32),
            pltpu.VMEM((ea,), jnp.int32),
            [pltpu.VMEM((BA, DH), jnp.float32) for _ in range(2)],
            [pltpu.VMEM((BA, 16), jnp.float32) for _ in range(2)],
            [pltpu.VMEM((BA, 16), jnp.float32) for _ in range(2)],
            [pltpu.VMEM((BA,), jnp.float32) for _ in range(2)],
            [pltpu.VMEM((BA,), jnp.int32) for _ in range(2)],
            pltpu.VMEM((16, DH), jnp.float32),
            [pltpu.SemaphoreType.DMA for _ in range(4)],
            [pltpu.SemaphoreType.DMA for _ in range(4)],
            [pltpu.SemaphoreType.DMA for _ in range(2)],
            pltpu.VMEM_SHARED((NP16, DH), jnp.float32),
            pltpu.VMEM_SHARED((NP16, DH), jnp.float32),
            pltpu.VMEM_SHARED((NP16, 16), jnp.float32),
        ],
    )
    def k(x_hbm, er_hbm, ec_hbm, ew_hbm, dis_hbm, out_hbm,
          rbuf, cbuf, rows2, drb2, dcb2, ew2, sidx2, zrows,
          gs0, gs1, ss2, aggr_sh, xsh, dsh):
        c = lax.axis_index("c")
        s = lax.axis_index("s")
        zero16 = jnp.zeros((16,), jnp.float32)
        nb = ea // BA
        base = c * (E // NSC) + s * ea

        for t in range(5):
            pltpu.sync_copy(x_hbm.at[pl.ds(s * 640 + t * 128, 128)],
                            xsh.at[pl.ds(s * 640 + t * 128, 128)])
        pltpu.sync_copy(dis_hbm.at[pl.ds(s * 640, 640)],
                        dsh.at[pl.ds(s * 640, 640)])
        pltpu.sync_copy(er_hbm.at[pl.ds(base, ea)], rbuf)
        pltpu.sync_copy(ec_hbm.at[pl.ds(base, ea)], cbuf)

        def zrow(i, _):
            for cc in range(DH // 16):
                zrows[i, pl.ds(cc * 16, 16)] = zero16
            return 0
        lax.fori_loop(0, 16, zrow, 0)

        def zcp(t, _):
            pltpu.sync_copy(zrows, aggr_sh.at[pl.ds(s * 640 + t * 16, 16)])
            return 0
        lax.fori_loop(0, 40, zcp, 0)
        plsc.subcore_barrier()

        def issue(p, bb):
            g = (gs0, gs1)[p]
            pltpu.async_copy(xsh.at[cbuf.at[pl.ds(bb * BA, BA)]],
                             rows2[p], g[0])
            pltpu.async_copy(dsh.at[rbuf.at[pl.ds(bb * BA, BA)]],
                             drb2[p], g[1])
            pltpu.async_copy(dsh.at[cbuf.at[pl.ds(bb * BA, BA)]],
                             dcb2[p], g[2])
            pltpu.async_copy(ew_hbm.at[pl.ds(base + bb * BA, BA)],
                             ew2[p], g[3])

        def gwait(p):
            g = (gs0, gs1)[p]
            pltpu.make_async_copy(xsh.at[cbuf.at[pl.ds(0, BA)]],
                                  rows2[p], g[0]).wait()
            pltpu.make_async_copy(dsh.at[rbuf.at[pl.ds(0, BA)]],
                                  drb2[p], g[1]).wait()
            pltpu.make_async_copy(dsh.at[cbuf.at[pl.ds(0, BA)]],
                                  dcb2[p], g[2]).wait()
            pltpu.make_async_copy(ew_hbm.at[pl.ds(0, BA)],
                                  ew2[p], g[3]).wait()

        def swait(p):
            pltpu.make_async_copy(rows2[p], aggr_sh.at[sidx2[p]],
                                  ss2[p]).wait()

        def compute(p, bb):
            rows, drb, dcb, sidx, ewb = (rows2[p], drb2[p], dcb2[p],
                                         sidx2[p], ew2[p])

            def wgrp(g, _):
                ew_g = ewb[pl.ds(g * 16, 16)]
                sidx[pl.ds(g * 16, 16)] = rbuf[pl.ds(bb * BA + g * 16, 16)]
                for j2 in range(16):
                    j = g * 16 + j2
                    rep = _lane_perm(ew_g, jnp.full((16,), j2, jnp.int32))
                    wv = drb[j, :] * rep * dcb[j, :]
                    for cc in range(DH // 16):
                        rows[j, pl.ds(cc * 16, 16)] = (
                            rows[j, pl.ds(cc * 16, 16)] * wv)
                return 0
            lax.fori_loop(0, BA // 16, wgrp, 0)
            pltpu.async_copy(rows, aggr_sh.at[sidx], ss2[p], add=True)

        issue(0, 0)

        def agg(b, _):
            nxt = jnp.minimum(b + 1, nb - 1)

            @pl.when(lax.rem(b, 2) == 0)
            def _():
                @pl.when(b >= 1)
                def _():
                    swait(1)
                issue(1, nxt)
                gwait(0)
                compute(0, b)

            @pl.when(lax.rem(b, 2) == 1)
            def _():
                swait(0)
                issue(0, nxt)
                gwait(1)
                compute(1, b)
            return 0
        lax.fori_loop(0, nb, agg, 0)
        gwait(1)
        swait(0)
        plsc.subcore_barrier()

        def ecp(t, _):
            pltpu.sync_copy(aggr_sh.at[pl.ds(s * 640 + t * 16, 16)], zrows)
            pltpu.sync_copy(zrows, out_hbm.at[c, pl.ds(s * 640 + t * 16, 16)])
            return 0
        lax.fori_loop(0, 40, ecp, 0)

    return k(x_half_pad, edge_row, edge_col, edge_weight, dis2)


def _dense_body(x_ref, aggr_ref, pt_ref, scal_ref, xa_ref, xfn_ref, loss_ref):
    x = x_ref[...]
    gamma = scal_ref[0, 0]
    alpha = scal_ref[0, 1]
    aggr_lo = aggr_ref[0, 0, :N] + aggr_ref[0, 1, :N]
    aggr_hi = aggr_ref[1, 0, :N] + aggr_ref[1, 1, :N]
    aggr = jnp.concatenate([aggr_lo, aggr_hi], axis=1)
    xs = (1.0 - gamma) * x + gamma * aggr
    xn = xs / jnp.maximum(jnp.sqrt(jnp.sum(xs * xs, axis=1, keepdims=True)), 1e-12)
    ptv = pt_ref[...]
    pn = ptv / jnp.maximum(jnp.sqrt(jnp.sum(ptv * ptv, axis=1, keepdims=True)), 1e-12)
    C = 1.0 - jax.lax.dot_general(xn, pn, (((1,), (1,)), ((), ())),
                                  preferred_element_type=jnp.float32)
    K = jnp.exp(-C / OT_EPS)
    a = 1.0 / N
    b = 1.0 / P

    def body(_, uv):
        u, v = uv
        u = a / (jnp.sum(K * v, axis=1, keepdims=True) + 1e-9)
        v = b / (jnp.sum(K * u, axis=0, keepdims=True) + 1e-9)
        return (u, v)

    u0 = jnp.full((N, 1), 1.0 / N, dtype=jnp.float32)
    v0 = jnp.full((1, P), 1.0 / P, dtype=jnp.float32)
    u, v = jax.lax.fori_loop(0, OT_ITERS, body, (u0, v0))
    T = u * K * v
    loss_ref[...] = jnp.sum(T * C)[None, None]
    pm = jax.lax.dot_general(T, ptv, (((1,), (0,)), ((), ())),
                             preferred_element_type=jnp.float32)
    xa = x + alpha * (float(N) * pm)
    xa_ref[...] = xa
    nrm = jnp.maximum(jnp.sqrt(jnp.sum(xa * xa, axis=1, keepdims=True)), 1e-12)
    xfn_ref[...] = xa / nrm


def _dense_call(x, aggr2, prompt_tokens, gamma, alpha):
    scal = jnp.stack([gamma, alpha]).reshape(1, 2).astype(jnp.float32)
    return pl.pallas_call(
        _dense_body,
        out_shape=[
            jax.ShapeDtypeStruct((N, D), jnp.float32),
            jax.ShapeDtypeStruct((N, D), jnp.float32),
            jax.ShapeDtypeStruct((1, 1), jnp.float32),
        ],
    )(x, aggr2, prompt_tokens, scal)


def _sc_dots_call(xfn_pad, pt_row, pt_col):
    ep = EPT_PAD // (NSC * NT)
    SEG = 3200
    B = 64

    mesh = plsc.VectorSubcoreMesh(core_axis_name="c", subcore_axis_name="s")

    @functools.partial(
        pl.kernel,
        out_type=jax.ShapeDtypeStruct((EPT_PAD,), jnp.float32),
        mesh=mesh,
        compiler_params=pltpu.CompilerParams(use_tc_tiling_on_sc=False),
        scratch_types=[
            pltpu.VMEM((SEG,), jnp.int32),
            pltpu.VMEM((SEG,), jnp.int32),
            [pltpu.VMEM((B, D), jnp.float32) for _ in range(2)],
            [pltpu.VMEM((B, D), jnp.float32) for _ in range(2)],
            pltpu.VMEM((SEG,), jnp.float32),
            [pltpu.SemaphoreType.DMA for _ in range(2)],
            [pltpu.SemaphoreType.DMA for _ in range(2)],
            pltpu.VMEM_SHARED((NP16, D), jnp.float32),
        ],
    )
    def k(x_hbm, pr_hbm, pc_hbm, out_hbm, rbuf, cbuf, ab2, bb2, obuf,
          sa2, sb2, xsh):
        c = lax.axis_index("c")
        s = lax.axis_index("s")
        w = s * NSC + c
        base = w * ep
        nb = SEG // B
        for t in range(5):
            pltpu.sync_copy(x_hbm.at[pl.ds(s * 640 + t * 128, 128)],
                            xsh.at[pl.ds(s * 640 + t * 128, 128)])
        plsc.subcore_barrier()
        iota = lax.broadcasted_iota(jnp.int32, (16,), 0)

        def issue(p, bb):
            pltpu.async_copy(xsh.at[rbuf.at[pl.ds(bb * B, B)]],
                             ab2[p], sa2[p])
            pltpu.async_copy(xsh.at[cbuf.at[pl.ds(bb * B, B)]],
                             bb2[p], sb2[p])

        def gwait(p):
            pltpu.make_async_copy(xsh.at[rbuf.at[pl.ds(0, B)]],
                                  ab2[p], sa2[p]).wait()
            pltpu.make_async_copy(xsh.at[cbuf.at[pl.ds(0, B)]],
                                  bb2[p], sb2[p]).wait()

        def compute(p, bb):
            abuf, bbuf = ab2[p], bb2[p]

            def grp(g, _):
                out_v = jnp.zeros((16,), jnp.float32)
                for j2 in range(16):
                    j = g * 16 + j2
                    acc = jnp.zeros((16,), jnp.float32)
                    for cc in range(8):
                        sv = (abuf[j, pl.ds(cc * 16, 16)]
                              + bbuf[j, pl.ds(cc * 16, 16)])
                        acc = acc + sv * sv
                    for k2 in (1, 2, 4, 8):
                        acc = acc + _lane_perm(acc, jnp.bitwise_xor(iota, k2))
                    out_v = jnp.where(iota == j2, acc, out_v)
                obuf[pl.ds(bb * B + g * 16, 16)] = out_v * 0.5 - 1.0
                return 0
            lax.fori_loop(0, B // 16, grp, 0)

        def segloop(seg, _):
            segbase = base + seg * SEG
            pltpu.sync_copy(pr_hbm.at[pl.ds(segbase, SEG)], rbuf)
            pltpu.sync_copy(pc_hbm.at[pl.ds(segbase, SEG)], cbuf)
            issue(0, 0)

            def batch(b, _):
                nxt = jnp.minimum(b + 1, nb - 1)

                @pl.when(lax.rem(b, 2) == 0)
                def _():
                    issue(1, nxt)
                    gwait(0)
                    compute(0, b)

                @pl.when(lax.rem(b, 2) == 1)
                def _():
                    issue(0, nxt)
                    gwait(1)
                    compute(1, b)
                return 0
            lax.fori_loop(0, nb, batch, 0)
            gwait(0)
            pltpu.sync_copy(obuf, out_hbm.at[pl.ds(segbase, SEG)])
            return 0
        lax.fori_loop(0, ep // SEG, segloop, 0)

    return k(xfn_pad, pt_row, pt_col)


def _ptw_body(d_ref, o_ref):
    d = d_ref[...]
    r = jnp.maximum(d, 0.0)
    rows = lax.broadcasted_iota(jnp.int32, (EPT_PAD // 128, 128), 0)
    cols = lax.broadcasted_iota(jnp.int32, (EPT_PAD // 128, 128), 1)
    valid = (rows * 128 + cols) < EPT
    r = jnp.where(valid, r, 0.0)
    m = jnp.max(r)
    o_ref[...] = r / (m + 1e-8)


def _ptw_call(dots2d):
    return pl.pallas_call(
        _ptw_body,
        out_shape=jax.ShapeDtypeStruct((EPT_PAD // 128, 128), jnp.float32),
    )(dots2d)


def kernel(x, edge_index, edge_weight, pt_edge_index, prompt_tokens, alpha_feat, gamma):
    er, ec = edge_index[0], edge_index[1]
    deg_parts = _sc_deg_call(ec)
    dis2 = _dis_call(deg_parts)
    x_pad = jnp.pad(x, ((0, NP16 - N), (0, 0)))
    aggr_lo = _sc_aggr_call(x_pad[:, :D // 2], er, ec, edge_weight, dis2)
    aggr_hi = _sc_aggr_call(x_pad[:, D // 2:], er, ec, edge_weight, dis2)
    aggr4 = jnp.stack([aggr_lo, aggr_hi])
    x_adapted, xfn, loss = _dense_call(x, aggr4, prompt_tokens, gamma, alpha_feat)
    pt_pad = jnp.pad(pt_edge_index, ((0, 0), (0, EPT_PAD - EPT)))
    xfn_pad = jnp.pad(xfn, ((0, NP16 - N), (0, 0)))
    dots = _sc_dots_call(xfn_pad, pt_pad[0], pt_pad[1])
    ptw = _ptw_call(dots.reshape(EPT_PAD // 128, 128)).reshape(EPT_PAD)[:EPT]
    return (x_adapted, loss[0, 0], pt_edge_index, ptw)


# speedup vs baseline: 1.0425x; 1.0425x over previous
import functools

import jax
import jax.numpy as jnp
from jax import lax
from jax.experimental import pallas as pl
from jax.experimental.pallas import tpu as pltpu
from jax.experimental.pallas import tpu_sc as plsc

N = 10000
NPR = 640
NP16 = NPR * 16
D = 128
P = 32
E = 320000
EPT = 500000
EPT_PAD = 512000
OT_EPS = 0.1
OT_ITERS = 20
NSC = 2
NT = 16

_GDN = lax.GatherDimensionNumbers(
    offset_dims=(), collapsed_slice_dims=(0,), start_index_map=(0,))


def _lane_perm(v, idx):
    return lax.gather(v, idx[:, None], _GDN, (1,),
                      mode=lax.GatherScatterMode.PROMISE_IN_BOUNDS)


def _sc_deg_call(edge_col):
    ed = E // (NSC * NT)
    NBD = ed // 128
    TAIL = ed - NBD * 128

    mesh = plsc.VectorSubcoreMesh(core_axis_name="c", subcore_axis_name="s")

    @functools.partial(
        pl.kernel,
        out_type=jax.ShapeDtypeStruct((NSC, NP16, 16), jnp.float32),
        mesh=mesh,
        compiler_params=pltpu.CompilerParams(use_tc_tiling_on_sc=False),
        scratch_types=[
            pltpu.VMEM((ed,), jnp.int32),
            pltpu.VMEM((128, 16), jnp.float32),
            pltpu.VMEM((128,), jnp.int32),
            pltpu.VMEM((TAIL,), jnp.int32),
            pltpu.VMEM((NPR, 16), jnp.float32),
            pltpu.VMEM_SHARED((NP16, 16), jnp.float32),
        ],
    )
    def k(ec_hbm, out_hbm, cbufd, ones_rows, degidx, degidxt, bnc, deg2_sh):
        c = lax.axis_index("c")
        s = lax.axis_index("s")
        zero16 = jnp.zeros((16,), jnp.float32)
        ones16 = jnp.ones((16,), jnp.float32)

        def zrow(i, _):
            ones_rows[i, :] = ones16
            bnc[lax.rem(i, NPR), :] = zero16
            return 0
        lax.fori_loop(0, 128, zrow, 0)

        def zb(i, _):
            bnc[i, :] = zero16
            return 0
        lax.fori_loop(0, NPR, zb, 0)
        pltpu.sync_copy(bnc, deg2_sh.at[pl.ds(s * NPR, NPR)])
        plsc.subcore_barrier()

        base = c * (E // NSC) + s * ed
        pltpu.sync_copy(ec_hbm.at[pl.ds(base, ed)], cbufd)

        def degb(i, _):
            for g in range(8):
                degidx[pl.ds(g * 16, 16)] = cbufd[pl.ds(i * 128 + g * 16, 16)]
            pltpu.sync_copy(ones_rows, deg2_sh.at[degidx], add=True)
            return 0
        lax.fori_loop(0, NBD, degb, 0)
        for g in range(TAIL // 16):
            degidxt[pl.ds(g * 16, 16)] = cbufd[pl.ds(NBD * 128 + g * 16, 16)]
        pltpu.sync_copy(ones_rows.at[pl.ds(0, TAIL)], deg2_sh.at[degidxt],
                        add=True)
        plsc.subcore_barrier()

        pltpu.sync_copy(deg2_sh.at[pl.ds(s * NPR, NPR)], bnc)
        pltpu.sync_copy(bnc, out_hbm.at[c, pl.ds(s * NPR, NPR)])

    return k(edge_col)


def _dis_body(dp_ref, dis_ref):
    deg = dp_ref[0] + dp_ref[1]
    dis_ref[...] = jnp.where(deg > 0.0, jax.lax.rsqrt(jnp.maximum(deg, 1.0)), 0.0)


def _dis_call(deg_parts):
    return pl.pallas_call(
        _dis_body,
        out_shape=jax.ShapeDtypeStruct((NP16, 16), jnp.float32),
    )(deg_parts)


def _sc_aggr_call(x_half, edge_row, edge_col, edge_weight, dis2):
    DH = D // 2
    ea = E // (NSC * NT)
    BA = 80

    mesh = plsc.VectorSubcoreMesh(core_axis_name="c", subcore_axis_name="s")

    @functools.partial(
        pl.kernel,
        out_type=jax.ShapeDtypeStruct((NSC, NP16, DH), jnp.float32),
        mesh=mesh,
        compiler_params=pltpu.CompilerParams(use_tc_tiling_on_sc=False),
        scratch_types=[
            pltpu.VMEM((ea,), jnp.int32),
            pltpu.VMEM((ea,), jnp.int32),
            pltpu.VMEM((ea,), jnp.float32),
            [pltpu.VMEM((BA, DH), jnp.float32) for _ in range(2)],
            [pltpu.VMEM((BA, 16), jnp.float32) for _ in range(2)],
            [pltpu.VMEM((BA, 16), jnp.float32) for _ in range(2)],
            [pltpu.VMEM((BA,), jnp.int32) for _ in range(2)],
            pltpu.VMEM((128, DH), jnp.float32),
            [pltpu.SemaphoreType.DMA for _ in range(3)],
            [pltpu.SemaphoreType.DMA for _ in range(3)],
            [pltpu.SemaphoreType.DMA for _ in range(2)],
            pltpu.VMEM_SHARED((NP16, DH), jnp.float32),
        ],
    )
    def k(x_hbm, er_hbm, ec_hbm, ew_hbm, dis_hbm, out_hbm,
          rbuf, cbuf, ewbuf, rows2, drb2, dcb2, sidx2, zrows,
          gs0, gs1, ss2, aggr_sh):
        c = lax.axis_index("c")
        s = lax.axis_index("s")
        zero16 = jnp.zeros((16,), jnp.float32)
        nb = ea // BA

        def zrow(i, _):
            for cc in range(DH // 16):
                zrows[i, pl.ds(cc * 16, 16)] = zero16
            return 0
        lax.fori_loop(0, 128, zrow, 0)
        for t in range(5):
            pltpu.sync_copy(zrows, aggr_sh.at[pl.ds(s * 640 + t * 128, 128)])
        plsc.subcore_barrier()

        base = c * (E // NSC) + s * ea
        pltpu.sync_copy(er_hbm.at[pl.ds(base, ea)], rbuf)
        pltpu.sync_copy(ec_hbm.at[pl.ds(base, ea)], cbuf)
        pltpu.sync_copy(ew_hbm.at[pl.ds(base, ea)], ewbuf)

        def issue(p, bb):
            pltpu.async_copy(x_hbm.at[cbuf.at[pl.ds(bb * BA, BA)]],
                             rows2[p], (gs0, gs1)[p][0])
            pltpu.async_copy(dis_hbm.at[rbuf.at[pl.ds(bb * BA, BA)]],
                             drb2[p], (gs0, gs1)[p][1])
            pltpu.async_copy(dis_hbm.at[cbuf.at[pl.ds(bb * BA, BA)]],
                             dcb2[p], (gs0, gs1)[p][2])

        def gwait(p):
            g = (gs0, gs1)[p]
            pltpu.make_async_copy(x_hbm.at[cbuf.at[pl.ds(0, BA)]],
                                  rows2[p], g[0]).wait()
            pltpu.make_async_copy(dis_hbm.at[rbuf.at[pl.ds(0, BA)]],
                                  drb2[p], g[1]).wait()
            pltpu.make_async_copy(dis_hbm.at[cbuf.at[pl.ds(0, BA)]],
                                  dcb2[p], g[2]).wait()

        def swait(p):
            pltpu.make_async_copy(rows2[p], aggr_sh.at[sidx2[p]],
                                  ss2[p]).wait()

        def compute(p, bb):
            rows, drb, dcb, sidx = rows2[p], drb2[p], dcb2[p], sidx2[p]

            def wgrp(g, _):
                ew_g = ewbuf[pl.ds(bb * BA + g * 16, 16)]
                sidx[pl.ds(g * 16, 16)] = rbuf[pl.ds(bb * BA + g * 16, 16)]
                for j2 in range(16):
                    j = g * 16 + j2
                    rep = _lane_perm(ew_g, jnp.full((16,), j2, jnp.int32))
                    wv = drb[j, :] * rep * dcb[j, :]
                    for cc in range(DH // 16):
                        rows[j, pl.ds(cc * 16, 16)] = (
                            rows[j, pl.ds(cc * 16, 16)] * wv)
                return 0
            lax.fori_loop(0, BA // 16, wgrp, 0)
            pltpu.async_copy(rows, aggr_sh.at[sidx], ss2[p], add=True)

        issue(0, 0)

        def agg(b, _):
            nxt = jnp.minimum(b + 1, nb - 1)

            @pl.when(lax.rem(b, 2) == 0)
            def _():
                @pl.when(b >= 1)
                def _():
                    swait(1)
                issue(1, nxt)
                gwait(0)
                compute(0, b)

            @pl.when(lax.rem(b, 2) == 1)
            def _():
                swait(0)
                issue(0, nxt)
                gwait(1)
                compute(1, b)
            return 0
        lax.fori_loop(0, nb, agg, 0)
        gwait(1)
        swait(0)
        plsc.subcore_barrier()

        for t in range(5):
            pltpu.sync_copy(aggr_sh.at[pl.ds(s * 640 + t * 128, 128)], zrows)
            pltpu.sync_copy(zrows, out_hbm.at[c, pl.ds(s * 640 + t * 128, 128)])

    return k(x_half, edge_row, edge_col, edge_weight, dis2)


def _dense_body(x_ref, alo_ref, ahi_ref, pt_ref, scal_ref, xa_ref, xfn_ref, loss_ref):
    x = x_ref[...]
    gamma = scal_ref[0, 0]
    alpha = scal_ref[0, 1]
    aggr_lo = alo_ref[0, :N] + alo_ref[1, :N]
    aggr_hi = ahi_ref[0, :N] + ahi_ref[1, :N]
    aggr = jnp.concatenate([aggr_lo, aggr_hi], axis=1)
    xs = (1.0 - gamma) * x + gamma * aggr
    xn = xs / jnp.maximum(jnp.sqrt(jnp.sum(xs * xs, axis=1, keepdims=True)), 1e-12)
    ptv = pt_ref[...]
    pn = ptv / jnp.maximum(jnp.sqrt(jnp.sum(ptv * ptv, axis=1, keepdims=True)), 1e-12)
    C = 1.0 - jax.lax.dot_general(xn, pn, (((1,), (1,)), ((), ())),
                                  preferred_element_type=jnp.float32)
    K = jnp.exp(-C / OT_EPS)
    a = 1.0 / N
    b = 1.0 / P

    def body(_, uv):
        u, v = uv
        u = a / (jnp.sum(K * v, axis=1, keepdims=True) + 1e-9)
        v = b / (jnp.sum(K * u, axis=0, keepdims=True) + 1e-9)
        return (u, v)

    u0 = jnp.full((N, 1), 1.0 / N, dtype=jnp.float32)
    v0 = jnp.full((1, P), 1.0 / P, dtype=jnp.float32)
    u, v = jax.lax.fori_loop(0, OT_ITERS, body, (u0, v0))
    T = u * K * v
    loss_ref[...] = jnp.sum(T * C)[None, None]
    pm = jax.lax.dot_general(T, ptv, (((1,), (0,)), ((), ())),
                             preferred_element_type=jnp.float32)
    xa = x + alpha * (float(N) * pm)
    xa_ref[...] = xa
    nrm = jnp.maximum(jnp.sqrt(jnp.sum(xa * xa, axis=1, keepdims=True)), 1e-12)
    xfn_ref[:N, :] = xa / nrm
    xfn_ref[N:, :] = jnp.zeros((NP16 - N, D), jnp.float32)


def _dense_call(x, aggr_lo, aggr_hi, prompt_tokens, gamma, alpha):
    scal = jnp.stack([gamma, alpha]).reshape(1, 2).astype(jnp.float32)
    return pl.pallas_call(
        _dense_body,
        out_shape=[
            jax.ShapeDtypeStruct((N, D), jnp.float32),
            jax.ShapeDtypeStruct((NP16, D), jnp.float32),
            jax.ShapeDtypeStruct((1, 1), jnp.float32),
        ],
    )(x, aggr_lo, aggr_hi, prompt_tokens, scal)


def _sc_dots_call(xfn_pad, pt_row, pt_col):
    ep = EPT_PAD // (NSC * NT)
    SEG = 3200
    B = 64

    mesh = plsc.VectorSubcoreMesh(core_axis_name="c", subcore_axis_name="s")

    @functools.partial(
        pl.kernel,
        out_type=jax.ShapeDtypeStruct((EPT_PAD,), jnp.float32),
        mesh=mesh,
        compiler_params=pltpu.CompilerParams(use_tc_tiling_on_sc=False),
        scratch_types=[
            pltpu.VMEM((SEG,), jnp.int32),
            pltpu.VMEM((SEG,), jnp.int32),
            [pltpu.VMEM((B, D), jnp.float32) for _ in range(2)],
            [pltpu.VMEM((B, D), jnp.float32) for _ in range(2)],
            pltpu.VMEM((SEG,), jnp.float32),
            [pltpu.SemaphoreType.DMA for _ in range(2)],
            [pltpu.SemaphoreType.DMA for _ in range(2)],
            pltpu.VMEM_SHARED((NP16, D), jnp.float32),
        ],
    )
    def k(x_hbm, pr_hbm, pc_hbm, out_hbm, rbuf, cbuf, ab2, bb2, obuf,
          sa2, sb2, xsh):
        c = lax.axis_index("c")
        s = lax.axis_index("s")
        w = s * NSC + c
        base = w * ep
        nb = SEG // B
        for t in range(5):
            pltpu.sync_copy(x_hbm.at[pl.ds(s * 640 + t * 128, 128)],
                            xsh.at[pl.ds(s * 640 + t * 128, 128)])
        plsc.subcore_barrier()
        iota = lax.broadcasted_iota(jnp.int32, (16,), 0)

        def issue(p, bb):
            pltpu.async_copy(xsh.at[rbuf.at[pl.ds(bb * B, B)]],
                             ab2[p], sa2[p])
            pltpu.async_copy(xsh.at[cbuf.at[pl.ds(bb * B, B)]],
                             bb2[p], sb2[p])

        def gwait(p):
            pltpu.make_async_copy(xsh.at[rbuf.at[pl.ds(0, B)]],
                                  ab2[p], sa2[p]).wait()
            pltpu.make_async_copy(xsh.at[cbuf.at[pl.ds(0, B)]],
                                  bb2[p], sb2[p]).wait()

        def compute(p, bb):
            abuf, bbuf = ab2[p], bb2[p]

            def grp(g, _):
                out_v = jnp.zeros((16,), jnp.float32)
                for j2 in range(16):
                    j = g * 16 + j2
                    acc = jnp.zeros((16,), jnp.float32)
                    for cc in range(8):
                        sv = (abuf[j, pl.ds(cc * 16, 16)]
                              + bbuf[j, pl.ds(cc * 16, 16)])
                        acc = acc + sv * sv
                    for k2 in (1, 2, 4, 8):
                        acc = acc + _lane_perm(acc, jnp.bitwise_xor(iota, k2))
                    out_v = jnp.where(iota == j2, acc, out_v)
                obuf[pl.ds(bb * B + g * 16, 16)] = out_v * 0.5 - 1.0
                return 0
            lax.fori_loop(0, B // 16, grp, 0)

        def segloop(seg, _):
            segbase = base + seg * SEG
            pltpu.sync_copy(pr_hbm.at[pl.ds(segbase, SEG)], rbuf)
            pltpu.sync_copy(pc_hbm.at[pl.ds(segbase, SEG)], cbuf)
            issue(0, 0)

            def batch(b, _):
                nxt = jnp.minimum(b + 1, nb - 1)

                @pl.when(lax.rem(b, 2) == 0)
                def _():
                    issue(1, nxt)
                    gwait(0)
                    compute(0, b)

                @pl.when(lax.rem(b, 2) == 1)
                def _():
                    issue(0, nxt)
                    gwait(1)
                    compute(1, b)
                return 0
            lax.fori_loop(0, nb, batch, 0)
            gwait(0)
            pltpu.sync_copy(obuf, out_hbm.at[pl.ds(segbase, SEG)])
            return 0
        lax.fori_loop(0, ep // SEG, segloop, 0)

    return k(xfn_pad, pt_row, pt_col)


def _ptw_body(d_ref, o_ref):
    d = d_ref[...]
    r = jnp.maximum(d, 0.0)
    rows = lax.broadcasted_iota(jnp.int32, (EPT_PAD // 128, 128), 0)
    cols = lax.broadcasted_iota(jnp.int32, (EPT_PAD // 128, 128), 1)
    valid = (rows * 128 + cols) < EPT
    r = jnp.where(valid, r, 0.0)
    m = jnp.max(r)
    o_ref[...] = r / (m + 1e-8)


def _ptw_call(dots2d):
    return pl.pallas_call(
        _ptw_body,
        out_shape=jax.ShapeDtypeStruct((EPT_PAD // 128, 128), jnp.float32),
    )(dots2d)


def kernel(x, edge_index, edge_weight, pt_edge_index, prompt_tokens, alpha_feat, gamma):
    er, ec = edge_index[0], edge_index[1]
    deg_parts = _sc_deg_call(ec)
    dis2 = _dis_call(deg_parts)
    aggr_lo = _sc_aggr_call(x[:, :D // 2], er, ec, edge_weight, dis2)
    aggr_hi = _sc_aggr_call(x[:, D // 2:], er, ec, edge_weight, dis2)
    x_adapted, xfn_pad, loss = _dense_call(x, aggr_lo, aggr_hi, prompt_tokens,
                                           gamma, alpha_feat)
    pt_pad = jnp.pad(pt_edge_index, ((0, 0), (0, EPT_PAD - EPT)))
    dots = _sc_dots_call(xfn_pad, pt_pad[0], pt_pad[1])
    ptw = _ptw_call(dots.reshape(EPT_PAD // 128, 128)).reshape(EPT_PAD)[:EPT]
    return (x_adapted, loss[0, 0], pt_edge_index, ptw)


# speedup vs baseline: 1.1912x; 1.1427x over previous
import functools

import jax
import jax.numpy as jnp
from jax import lax
from jax.experimental import pallas as pl
from jax.experimental.pallas import tpu as pltpu
from jax.experimental.pallas import tpu_sc as plsc

N = 10000
NPR = 640
NP16 = NPR * 16
D = 128
P = 32
E = 320000
EPT = 500000
EPT_PAD = 512000
OT_EPS = 0.1
OT_ITERS = 20
NSC = 2
NT = 16

_GDN = lax.GatherDimensionNumbers(
    offset_dims=(), collapsed_slice_dims=(0,), start_index_map=(0,))


def _lane_perm(v, idx):
    return lax.gather(v, idx[:, None], _GDN, (1,),
                      mode=lax.GatherScatterMode.PROMISE_IN_BOUNDS)


def _sc_deg_call(edge_col):
    ed = E // (NSC * NT)
    NBD = ed // 128
    TAIL = ed - NBD * 128

    mesh = plsc.VectorSubcoreMesh(core_axis_name="c", subcore_axis_name="s")

    @functools.partial(
        pl.kernel,
        out_type=jax.ShapeDtypeStruct((NSC, NP16, 16), jnp.float32),
        mesh=mesh,
        compiler_params=pltpu.CompilerParams(use_tc_tiling_on_sc=False),
        scratch_types=[
            pltpu.VMEM((ed,), jnp.int32),
            pltpu.VMEM((128, 16), jnp.float32),
            pltpu.VMEM((128,), jnp.int32),
            pltpu.VMEM((TAIL,), jnp.int32),
            pltpu.VMEM((NPR, 16), jnp.float32),
            pltpu.VMEM_SHARED((NP16, 16), jnp.float32),
        ],
    )
    def k(ec_hbm, out_hbm, cbufd, ones_rows, degidx, degidxt, bnc, deg2_sh):
        c = lax.axis_index("c")
        s = lax.axis_index("s")
        zero16 = jnp.zeros((16,), jnp.float32)
        ones16 = jnp.ones((16,), jnp.float32)

        def zrow(i, _):
            ones_rows[i, :] = ones16
            bnc[lax.rem(i, NPR), :] = zero16
            return 0
        lax.fori_loop(0, 128, zrow, 0)

        def zb(i, _):
            bnc[i, :] = zero16
            return 0
        lax.fori_loop(0, NPR, zb, 0)
        pltpu.sync_copy(bnc, deg2_sh.at[pl.ds(s * NPR, NPR)])
        plsc.subcore_barrier()

        base = c * (E // NSC) + s * ed
        pltpu.sync_copy(ec_hbm.at[pl.ds(base, ed)], cbufd)

        def degb(i, _):
            for g in range(8):
                degidx[pl.ds(g * 16, 16)] = cbufd[pl.ds(i * 128 + g * 16, 16)]
            pltpu.sync_copy(ones_rows, deg2_sh.at[degidx], add=True)
            return 0
        lax.fori_loop(0, NBD, degb, 0)
        for g in range(TAIL // 16):
            degidxt[pl.ds(g * 16, 16)] = cbufd[pl.ds(NBD * 128 + g * 16, 16)]
        pltpu.sync_copy(ones_rows.at[pl.ds(0, TAIL)], deg2_sh.at[degidxt],
                        add=True)
        plsc.subcore_barrier()

        pltpu.sync_copy(deg2_sh.at[pl.ds(s * NPR, NPR)], bnc)
        pltpu.sync_copy(bnc, out_hbm.at[c, pl.ds(s * NPR, NPR)])

    return k(edge_col)


def _dis_body(dp_ref, dis_ref):
    deg = dp_ref[0] + dp_ref[1]
    dis_ref[...] = jnp.where(deg > 0.0, jax.lax.rsqrt(jnp.maximum(deg, 1.0)), 0.0)


def _dis_call(deg_parts):
    return pl.pallas_call(
        _dis_body,
        out_shape=jax.ShapeDtypeStruct((NP16, 16), jnp.float32),
    )(deg_parts)


def _sc_aggr_call(x_half, edge_row, edge_col, edge_weight, dis2):
    DH = D // 2
    ea = E // (NSC * NT)
    BA = 80

    mesh = plsc.VectorSubcoreMesh(core_axis_name="c", subcore_axis_name="s")

    @functools.partial(
        pl.kernel,
        out_type=jax.ShapeDtypeStruct((NSC, NP16, DH), jnp.float32),
        mesh=mesh,
        compiler_params=pltpu.CompilerParams(use_tc_tiling_on_sc=False),
        scratch_types=[
            pltpu.VMEM((ea,), jnp.int32),
            pltpu.VMEM((ea,), jnp.int32),
            pltpu.VMEM((ea,), jnp.float32),
            [pltpu.VMEM((BA, DH), jnp.float32) for _ in range(2)],
            [pltpu.VMEM((BA, 16), jnp.float32) for _ in range(2)],
            [pltpu.VMEM((BA, 16), jnp.float32) for _ in range(2)],
            [pltpu.VMEM((BA,), jnp.int32) for _ in range(2)],
            pltpu.VMEM((128, DH), jnp.float32),
            [pltpu.SemaphoreType.DMA for _ in range(3)],
            [pltpu.SemaphoreType.DMA for _ in range(3)],
            [pltpu.SemaphoreType.DMA for _ in range(2)],
            pltpu.VMEM_SHARED((NP16, DH), jnp.float32),
        ],
    )
    def k(x_hbm, er_hbm, ec_hbm, ew_hbm, dis_hbm, out_hbm,
          rbuf, cbuf, ewbuf, rows2, drb2, dcb2, sidx2, zrows,
          gs0, gs1, ss2, aggr_sh):
        c = lax.axis_index("c")
        s = lax.axis_index("s")
        zero16 = jnp.zeros((16,), jnp.float32)
        nb = ea // BA

        def zrow(i, _):
            for cc in range(DH // 16):
                zrows[i, pl.ds(cc * 16, 16)] = zero16
            return 0
        lax.fori_loop(0, 128, zrow, 0)
        for t in range(5):
            pltpu.sync_copy(zrows, aggr_sh.at[pl.ds(s * 640 + t * 128, 128)])
        plsc.subcore_barrier()

        base = c * (E // NSC) + s * ea
        pltpu.sync_copy(er_hbm.at[pl.ds(base, ea)], rbuf)
        pltpu.sync_copy(ec_hbm.at[pl.ds(base, ea)], cbuf)
        pltpu.sync_copy(ew_hbm.at[pl.ds(base, ea)], ewbuf)

        def issue(p, bb):
            pltpu.async_copy(x_hbm.at[cbuf.at[pl.ds(bb * BA, BA)]],
                             rows2[p], (gs0, gs1)[p][0])
            pltpu.async_copy(dis_hbm.at[rbuf.at[pl.ds(bb * BA, BA)]],
                             drb2[p], (gs0, gs1)[p][1])
            pltpu.async_copy(dis_hbm.at[cbuf.at[pl.ds(bb * BA, BA)]],
                             dcb2[p], (gs0, gs1)[p][2])

        def gwait(p):
            g = (gs0, gs1)[p]
            pltpu.make_async_copy(x_hbm.at[cbuf.at[pl.ds(0, BA)]],
                                  rows2[p], g[0]).wait()
            pltpu.make_async_copy(dis_hbm.at[rbuf.at[pl.ds(0, BA)]],
                                  drb2[p], g[1]).wait()
            pltpu.make_async_copy(dis_hbm.at[cbuf.at[pl.ds(0, BA)]],
                                  dcb2[p], g[2]).wait()

        def swait(p):
            pltpu.make_async_copy(rows2[p], aggr_sh.at[sidx2[p]],
                                  ss2[p]).wait()

        def compute(p, bb):
            rows, drb, dcb, sidx = rows2[p], drb2[p], dcb2[p], sidx2[p]

            def wgrp(g, _):
                ew_g = ewbuf[pl.ds(bb * BA + g * 16, 16)]
                sidx[pl.ds(g * 16, 16)] = rbuf[pl.ds(bb * BA + g * 16, 16)]
                for j2 in range(16):
                    j = g * 16 + j2
                    rep = _lane_perm(ew_g, jnp.full((16,), j2, jnp.int32))
                    wv = drb[j, :] * rep * dcb[j, :]
                    for cc in range(DH // 16):
                        rows[j, pl.ds(cc * 16, 16)] = (
                            rows[j, pl.ds(cc * 16, 16)] * wv)
                return 0
            lax.fori_loop(0, BA // 16, wgrp, 0)
            pltpu.async_copy(rows, aggr_sh.at[sidx], ss2[p], add=True)

        issue(0, 0)

        def agg(b, _):
            nxt = jnp.minimum(b + 1, nb - 1)

            @pl.when(lax.rem(b, 2) == 0)
            def _():
                @pl.when(b >= 1)
                def _():
                    swait(1)
                issue(1, nxt)
                gwait(0)
                compute(0, b)

            @pl.when(lax.rem(b, 2) == 1)
            def _():
                swait(0)
                issue(0, nxt)
                gwait(1)
                compute(1, b)
            return 0
        lax.fori_loop(0, nb, agg, 0)
        gwait(1)
        swait(0)
        plsc.subcore_barrier()

        for t in range(5):
            pltpu.sync_copy(aggr_sh.at[pl.ds(s * 640 + t * 128, 128)], zrows)
            pltpu.sync_copy(zrows, out_hbm.at[c, pl.ds(s * 640 + t * 128, 128)])

    return k(x_half, edge_row, edge_col, edge_weight, dis2)


def _dense_body(x_ref, alo_ref, ahi_ref, pt_ref, scal_ref, xa_ref, xfn_ref, loss_ref):
    x = x_ref[...]
    gamma = scal_ref[0, 0]
    alpha = scal_ref[0, 1]
    aggr_lo = alo_ref[0, :N] + alo_ref[1, :N]
    aggr_hi = ahi_ref[0, :N] + ahi_ref[1, :N]
    aggr = jnp.concatenate([aggr_lo, aggr_hi], axis=1)
    xs = (1.0 - gamma) * x + gamma * aggr
    xn = xs / jnp.maximum(jnp.sqrt(jnp.sum(xs * xs, axis=1, keepdims=True)), 1e-12)
    ptv = pt_ref[...]
    pn = ptv / jnp.maximum(jnp.sqrt(jnp.sum(ptv * ptv, axis=1, keepdims=True)), 1e-12)
    C = 1.0 - jax.lax.dot_general(xn, pn, (((1,), (1,)), ((), ())),
                                  preferred_element_type=jnp.float32)
    K = jnp.exp(-C / OT_EPS)
    a = 1.0 / N
    b = 1.0 / P

    def body(_, uv):
        u, v = uv
        u = a / (jnp.sum(K * v, axis=1, keepdims=True) + 1e-9)
        v = b / (jnp.sum(K * u, axis=0, keepdims=True) + 1e-9)
        return (u, v)

    u0 = jnp.full((N, 1), 1.0 / N, dtype=jnp.float32)
    v0 = jnp.full((1, P), 1.0 / P, dtype=jnp.float32)
    u, v = jax.lax.fori_loop(0, OT_ITERS, body, (u0, v0))
    T = u * K * v
    loss_ref[...] = jnp.sum(T * C)[None, None]
    pm = jax.lax.dot_general(T, ptv, (((1,), (0,)), ((), ())),
                             preferred_element_type=jnp.float32)
    xa = x + alpha * (float(N) * pm)
    xa_ref[...] = xa
    nrm = jnp.maximum(jnp.sqrt(jnp.sum(xa * xa, axis=1, keepdims=True)), 1e-12)
    xfn_ref[:N, :] = xa / nrm
    xfn_ref[N:, :] = jnp.zeros((NP16 - N, D), jnp.float32)


def _dense_call(x, aggr_lo, aggr_hi, prompt_tokens, gamma, alpha):
    scal = jnp.stack([gamma, alpha]).reshape(1, 2).astype(jnp.float32)
    return pl.pallas_call(
        _dense_body,
        out_shape=[
            jax.ShapeDtypeStruct((N, D), jnp.float32),
            jax.ShapeDtypeStruct((NP16, D), jnp.float32),
            jax.ShapeDtypeStruct((1, 1), jnp.float32),
        ],
    )(x, aggr_lo, aggr_hi, prompt_tokens, scal)


def _sc_dots_call(xfn_pad, pt_row, pt_col):
    ep = EPT_PAD // (NSC * NT)
    SEG = 640
    B = 64

    mesh = plsc.VectorSubcoreMesh(core_axis_name="c", subcore_axis_name="s")

    @functools.partial(
        pl.kernel,
        out_type=jax.ShapeDtypeStruct((EPT_PAD, 16), jnp.float32),
        mesh=mesh,
        compiler_params=pltpu.CompilerParams(use_tc_tiling_on_sc=False),
        scratch_types=[
            pltpu.VMEM((SEG,), jnp.int32),
            pltpu.VMEM((SEG,), jnp.int32),
            [pltpu.VMEM((B, D), jnp.float32) for _ in range(2)],
            [pltpu.VMEM((B, D), jnp.float32) for _ in range(2)],
            pltpu.VMEM((SEG, 16), jnp.float32),
            [pltpu.SemaphoreType.DMA for _ in range(2)],
            [pltpu.SemaphoreType.DMA for _ in range(2)],
            pltpu.VMEM_SHARED((NP16, D), jnp.float32),
        ],
    )
    def k(x_hbm, pr_hbm, pc_hbm, out_hbm, rbuf, cbuf, ab2, bb2, obuf,
          sa2, sb2, xsh):
        c = lax.axis_index("c")
        s = lax.axis_index("s")
        w = s * NSC + c
        base = w * ep
        nb = SEG // B
        for t in range(5):
            pltpu.sync_copy(x_hbm.at[pl.ds(s * 640 + t * 128, 128)],
                            xsh.at[pl.ds(s * 640 + t * 128, 128)])
        plsc.subcore_barrier()
        iota = lax.broadcasted_iota(jnp.int32, (16,), 0)

        def issue(p, bb):
            pltpu.async_copy(xsh.at[rbuf.at[pl.ds(bb * B, B)]],
                             ab2[p], sa2[p])
            pltpu.async_copy(xsh.at[cbuf.at[pl.ds(bb * B, B)]],
                             bb2[p], sb2[p])

        def gwait(p):
            pltpu.make_async_copy(xsh.at[rbuf.at[pl.ds(0, B)]],
                                  ab2[p], sa2[p]).wait()
            pltpu.make_async_copy(xsh.at[cbuf.at[pl.ds(0, B)]],
                                  bb2[p], sb2[p]).wait()

        def compute(p, bb):
            abuf, bbuf = ab2[p], bb2[p]

            def grp(g, _):
                for j2 in range(16):
                    j = g * 16 + j2
                    acc = jnp.zeros((16,), jnp.float32)
                    for cc in range(8):
                        sv = (abuf[j, pl.ds(cc * 16, 16)]
                              + bbuf[j, pl.ds(cc * 16, 16)])
                        acc = acc + sv * sv
                    obuf[bb * B + j, :] = acc
                return 0
            lax.fori_loop(0, B // 16, grp, 0)

        def segloop(seg, _):
            segbase = base + seg * SEG
            pltpu.sync_copy(pr_hbm.at[pl.ds(segbase, SEG)], rbuf)
            pltpu.sync_copy(pc_hbm.at[pl.ds(segbase, SEG)], cbuf)
            issue(0, 0)

            def batch(b, _):
                nxt = jnp.minimum(b + 1, nb - 1)

                @pl.when(lax.rem(b, 2) == 0)
                def _():
                    issue(1, nxt)
                    gwait(0)
                    compute(0, b)

                @pl.when(lax.rem(b, 2) == 1)
                def _():
                    issue(0, nxt)
                    gwait(1)
                    compute(1, b)
                return 0
            lax.fori_loop(0, nb, batch, 0)
            gwait(0)
            pltpu.sync_copy(obuf, out_hbm.at[pl.ds(segbase, SEG)])
            return 0
        lax.fori_loop(0, ep // SEG, segloop, 0)

    return k(xfn_pad, pt_row, pt_col)


def _lsum_body(q_ref, o_ref):
    q = q_ref[...]
    lanes = lax.broadcasted_iota(jnp.int32, (128, 8), 0)
    sel = lax.broadcasted_iota(jnp.int32, (128, 8), 1)
    bsel = jnp.where(lanes // 16 == sel, 1.0, 0.0).astype(jnp.float32)
    s = jax.lax.dot_general(q, bsel, (((1,), (0,)), ((), ())),
                            preferred_element_type=jnp.float32)
    o_ref[...] = s * 0.5 - 1.0


def _lsum_call(q):
    nrow = EPT_PAD // 8
    blk = nrow // 16
    return pl.pallas_call(
        _lsum_body,
        grid=(16,),
        in_specs=[pl.BlockSpec((blk, 128), lambda i: (i, 0))],
        out_specs=pl.BlockSpec((blk, 8), lambda i: (i, 0)),
        out_shape=jax.ShapeDtypeStruct((nrow, 8), jnp.float32),
    )(q)


def _ptw_body(d_ref, o_ref):
    d = d_ref[...]
    r = jnp.maximum(d, 0.0)
    rows = lax.broadcasted_iota(jnp.int32, (EPT_PAD // 128, 128), 0)
    cols = lax.broadcasted_iota(jnp.int32, (EPT_PAD // 128, 128), 1)
    valid = (rows * 128 + cols) < EPT
    r = jnp.where(valid, r, 0.0)
    m = jnp.max(r)
    o_ref[...] = r / (m + 1e-8)


def _ptw_call(dots2d):
    return pl.pallas_call(
        _ptw_body,
        out_shape=jax.ShapeDtypeStruct((EPT_PAD // 128, 128), jnp.float32),
    )(dots2d)


def kernel(x, edge_index, edge_weight, pt_edge_index, prompt_tokens, alpha_feat, gamma):
    er, ec = edge_index[0], edge_index[1]
    deg_parts = _sc_deg_call(ec)
    dis2 = _dis_call(deg_parts)
    aggr_lo = _sc_aggr_call(x[:, :D // 2], er, ec, edge_weight, dis2)
    aggr_hi = _sc_aggr_call(x[:, D // 2:], er, ec, edge_weight, dis2)
    x_adapted, xfn_pad, loss = _dense_call(x, aggr_lo, aggr_hi, prompt_tokens,
                                           gamma, alpha_feat)
    pt_pad = jnp.pad(pt_edge_index, ((0, 0), (0, EPT_PAD - EPT)))
    q16 = _sc_dots_call(xfn_pad, pt_pad[0], pt_pad[1])
    dots = _lsum_call(q16.reshape(EPT_PAD // 8, 128))
    ptw = _ptw_call(dots.reshape(EPT_PAD // 128, 128)).reshape(EPT_PAD)[:EPT]
    return (x_adapted, loss[0, 0], pt_edge_index, ptw)
